# baseline (speedup 1.0000x reference)
import functools

import numpy as np
import jax
import jax.numpy as jnp
from jax import lax
from jax.experimental import pallas as pl
from jax.experimental.pallas import tpu as pltpu
from jax.experimental.pallas import tpu_sc as plsc

B = 4096
S = 200
D = 64
NW = 32
BPW = B // NW
EPS = 1e-12
NPH = 4


def _pos_table():
    dims = np.repeat(np.arange(D // 2), 2) * 2
    dims = 1.0 / np.power(10000, dims / D)
    enc = np.outer(np.arange(S), dims)
    enc[:, 0::2] = np.sin(enc[:, 0::2])
    enc[:, 1::2] = np.cos(enc[:, 1::2])
    return np.asarray(enc, dtype=np.float32)


_POS = _pos_table()


def _rsqrt(v):
    i = lax.bitcast_convert_type(v, jnp.int32)
    i = jnp.int32(0x5F3759DF) - lax.shift_right_logical(i, 1)
    y = lax.bitcast_convert_type(i, jnp.float32)
    half = v * 0.5
    for _ in range(3):
        y = y * (1.5 - half * y * y)
    return y


def _body(idx_hbm, table_hbm, gamma_hbm, beta_hbm, pos_hbm, out_hbm,
          idxbuf, posbuf, gbuf, bbuf, rbs, gsems, osems):
    wid = lax.axis_index("s") * 2 + lax.axis_index("c")
    b0 = wid * BPW

    pltpu.sync_copy(idx_hbm.at[pl.ds(b0 * S, BPW * S)], idxbuf)
    pltpu.sync_copy(pos_hbm, posbuf)
    pltpu.sync_copy(gamma_hbm, gbuf)
    pltpu.sync_copy(beta_hbm, bbuf)

    gks = [gbuf[pl.ds(16 * k, 16)] for k in range(4)]
    bks = [bbuf[pl.ds(16 * k, 16)] for k in range(4)]

    def gather_descs(i, ph):
        return (
            pltpu.make_async_copy(
                table_hbm.at[idxbuf.at[pl.ds(i * S, 128)]],
                rbs[ph].at[pl.ds(0, 128)], gsems[ph]),
            pltpu.make_async_copy(
                table_hbm.at[idxbuf.at[pl.ds(i * S + 128, 72)]],
                rbs[ph].at[pl.ds(128, 72)], gsems[ph]),
        )

    def out_desc(i, ph):
        return pltpu.make_async_copy(
            rbs[ph], out_hbm.at[pl.ds((b0 + i) * S, S)], osems[ph])

    def compute(ph):
        rb = rbs[ph]

        def row(r, carry):
            xs = [rb[r, pl.ds(16 * k, 16)] + posbuf[r, pl.ds(16 * k, 16)]
                  for k in range(4)]
            tot = jnp.sum((xs[0] + xs[1]) + (xs[2] + xs[3]))
            sq = [x * x for x in xs]
            tot2 = jnp.sum((sq[0] + sq[1]) + (sq[2] + sq[3]))
            mean = tot * (1.0 / D)
            var = tot2 * (1.0 / D) - mean * mean
            scale = _rsqrt(var + EPS)
            for k in range(4):
                rb[r, pl.ds(16 * k, 16)] = (xs[k] - mean) * scale * gks[k] + bks[k]
            return carry

        lax.fori_loop(0, S, row, 0)

    for d in gather_descs(0, 0):
        d.start()

    def step(i, ph):
        phn = (ph + 1) % NPH
        for d in gather_descs(i, ph):
            d.wait()

        @pl.when(i >= NPH - 1)
        def _():
            out_desc(i - (NPH - 1), phn).wait()

        @pl.when(i + 1 < BPW)
        def _():
            for d in gather_descs(i + 1, phn):
                d.start()

        compute(ph)
        out_desc(i, ph).start()

    def outer(j, carry):
        for ph in range(NPH):
            step(j * NPH + ph, ph)
        return carry

    lax.fori_loop(0, BPW // NPH, outer, 0)

    for off in range(NPH - 1, 0, -1):
        out_desc(BPW - off, (BPW - off) % NPH).wait()


def _sc_embed(idx_flat, table, gamma, beta, pos):
    mesh = plsc.VectorSubcoreMesh(core_axis_name="c", subcore_axis_name="s")
    f = pl.kernel(
        _body,
        out_type=jax.ShapeDtypeStruct((B * S, D), jnp.float32),
        mesh=mesh,
        compiler_params=pltpu.CompilerParams(
            needs_layout_passes=False, use_tc_tiling_on_sc=False),
        scratch_types=dict(
            idxbuf=pltpu.VMEM((BPW * S,), jnp.int32),
            posbuf=pltpu.VMEM((S, D), jnp.float32),
            gbuf=pltpu.VMEM((D,), jnp.float32),
            bbuf=pltpu.VMEM((D,), jnp.float32),
            rbs=[pltpu.VMEM((S, D), jnp.float32) for _ in range(NPH)],
            gsems=[pltpu.SemaphoreType.DMA for _ in range(NPH)],
            osems=[pltpu.SemaphoreType.DMA for _ in range(NPH)],
        ),
    )
    return f(idx_flat, table, gamma, beta, pos)


def kernel(inputs, table, gamma, beta):
    idx_flat = inputs.reshape(-1).astype(jnp.int32)
    out = _sc_embed(idx_flat, table, gamma, beta, jnp.asarray(_POS))
    return out.reshape(B, S, D)


# speedup vs baseline: 1.7656x; 1.7656x over previous
import numpy as np
import jax
import jax.numpy as jnp
from jax import lax
from jax.experimental import pallas as pl
from jax.experimental.pallas import tpu as pltpu
from jax.experimental.pallas import tpu_sc as plsc

B = 4096
S = 200
D = 64
NW = 32
BPW = B // NW
EPS = 1e-12
NPH = 4
UNROLL = 4


def _pos_table():
    dims = np.repeat(np.arange(D // 2), 2) * 2
    dims = 1.0 / np.power(10000, dims / D)
    enc = np.outer(np.arange(S), dims)
    enc[:, 0::2] = np.sin(enc[:, 0::2])
    enc[:, 1::2] = np.cos(enc[:, 1::2])
    return np.asarray(enc, dtype=np.float32)


_POS = _pos_table()


def _rsqrt(v):
    i = lax.bitcast_convert_type(v, jnp.int32)
    i = jnp.int32(0x5F3759DF) - lax.shift_right_logical(i, 1)
    y = lax.bitcast_convert_type(i, jnp.float32)
    half = v * 0.5
    for _ in range(2):
        y = y * (1.5 - half * y * y)
    return y


def _body(idx_hbm, table_hbm, gamma_hbm, beta_hbm, pos_hbm, out_hbm,
          idxbuf, posbuf, gbuf, bbuf, rbs, gsems, osems):
    wid = lax.axis_index("s") * 2 + lax.axis_index("c")
    b0 = wid * BPW

    pltpu.sync_copy(idx_hbm.at[pl.ds(b0, BPW)], idxbuf)
    pltpu.sync_copy(pos_hbm, posbuf)
    pltpu.sync_copy(gamma_hbm, gbuf)
    pltpu.sync_copy(beta_hbm, bbuf)

    gks = [gbuf[pl.ds(16 * k, 16)] for k in range(4)]
    bks = [bbuf[pl.ds(16 * k, 16)] for k in range(4)]

    def gather_descs(i, ph):
        return (
            pltpu.make_async_copy(
                table_hbm.at[idxbuf.at[i, pl.ds(0, 128)]],
                rbs[ph].at[pl.ds(0, 128)], gsems[ph]),
            pltpu.make_async_copy(
                table_hbm.at[idxbuf.at[i, pl.ds(128, 72)]],
                rbs[ph].at[pl.ds(128, 72)], gsems[ph]),
        )

    def out_desc(i, ph):
        return pltpu.make_async_copy(rbs[ph], out_hbm.at[b0 + i], osems[ph])

    def compute(ph):
        rb = rbs[ph]

        @plsc.parallel_loop(0, S, unroll=UNROLL)
        def row(r):
            xs = [rb[r, pl.ds(16 * k, 16)] + posbuf[r, pl.ds(16 * k, 16)]
                  for k in range(4)]
            tot = jnp.sum((xs[0] + xs[1]) + (xs[2] + xs[3]))
            sq = [x * x for x in xs]
            tot2 = jnp.sum((sq[0] + sq[1]) + (sq[2] + sq[3]))
            mean = tot * (1.0 / D)
            var = tot2 * (1.0 / D) - mean * mean
            scale = _rsqrt(var + EPS)
            for k in range(4):
                rb[r, pl.ds(16 * k, 16)] = (xs[k] - mean) * scale * gks[k] + bks[k]

    for d in gather_descs(0, 0):
        d.start()

    def step(i, ph):
        phn = (ph + 1) % NPH
        for d in gather_descs(i, ph):
            d.wait()

        @pl.when(i >= NPH - 1)
        def _():
            out_desc(i - (NPH - 1), phn).wait()

        @pl.when(i + 1 < BPW)
        def _():
            for d in gather_descs(i + 1, phn):
                d.start()

        compute(ph)
        out_desc(i, ph).start()

    def outer(j, carry):
        for ph in range(NPH):
            step(j * NPH + ph, ph)
        return carry

    lax.fori_loop(0, BPW // NPH, outer, 0)

    for off in range(NPH - 1, 0, -1):
        out_desc(BPW - off, (BPW - off) % NPH).wait()


def _sc_embed(idx, table, gamma, beta, pos):
    mesh = plsc.VectorSubcoreMesh(core_axis_name="c", subcore_axis_name="s")
    f = pl.kernel(
        _body,
        out_type=jax.ShapeDtypeStruct((B, S, D), jnp.float32),
        mesh=mesh,
        compiler_params=pltpu.CompilerParams(
            needs_layout_passes=False, use_tc_tiling_on_sc=False),
        scratch_types=dict(
            idxbuf=pltpu.VMEM((BPW, S), jnp.int32),
            posbuf=pltpu.VMEM((S, D), jnp.float32),
            gbuf=pltpu.VMEM((D,), jnp.float32),
            bbuf=pltpu.VMEM((D,), jnp.float32),
            rbs=[pltpu.VMEM((S, D), jnp.float32) for _ in range(NPH)],
            gsems=[pltpu.SemaphoreType.DMA for _ in range(NPH)],
            osems=[pltpu.SemaphoreType.DMA for _ in range(NPH)],
        ),
    )
    return f(idx, table, gamma, beta, pos)


def kernel(inputs, table, gamma, beta):
    return _sc_embed(inputs.astype(jnp.int32), table, gamma, beta,
                     jnp.asarray(_POS))
